# speedup vs baseline: 1.9111x; 1.9111x over previous
import jax
import jax.numpy as jnp
from jax.experimental import pallas as pl

D_MODEL = 1024
MAX_SEQ = 8192
SEP_ID = 102
S = 256
WR = S + 16


def _pos_kernel(tok_ref, pos_ref):
    m = (tok_ref[...] == SEP_ID).astype(jnp.int32)
    acc = m
    k = 1
    while k < MAX_SEQ:
        zeros = jnp.zeros((acc.shape[0], k), jnp.int32)
        acc = acc + jnp.concatenate([zeros, acc[:, :-k]], axis=1)
        k *= 2
    pos_ref[...] = acc


def _main_kernel(x_ref, pos_ref, pe_ref, o_ref):
    pos_v = pos_ref[0, 0]
    first = jnp.min(pos_v)
    last = jnp.max(pos_v)
    pos_c = jnp.minimum(pos_v, MAX_SEQ - 1)
    xb = x_ref[0]

    @pl.when(last == first)
    def _fast():
        p = jnp.minimum(first, MAX_SEQ - 1)
        p8 = (p // 8) * 8
        w8 = pe_ref[pl.ds(p8, 8), :]
        sel = (jax.lax.broadcasted_iota(jnp.int32, (8, 1), 0)
               == (p - p8)).astype(jnp.float32)
        row = jnp.sum(w8 * sel, axis=0, keepdims=True)
        o_ref[0] = xb + row

    @pl.when(last != first)
    def _general():
        base = jnp.minimum(first, MAX_SEQ - WR)
        base8 = (base // 8) * 8
        w = pe_ref[pl.ds(base8, WR), :]
        r = pos_c - base8
        oh = (jax.lax.broadcasted_iota(jnp.int32, (WR, S), 0)
              == jnp.broadcast_to(r, (WR, S))).astype(jnp.float32)
        y = jax.lax.dot_general(oh, w, (((0,), (0,)), ((), ())),
                                preferred_element_type=jnp.float32)
        o_ref[0] = xb + y


@jax.jit
def kernel(x, tokens, pe):
    B, L, D = x.shape
    nb = L // S
    pos = pl.pallas_call(
        _pos_kernel,
        out_shape=jax.ShapeDtypeStruct((B, L), jnp.int32),
    )(tokens)
    pos4 = pos.reshape(B, nb, 1, S)
    out = pl.pallas_call(
        _main_kernel,
        grid=(B, nb),
        in_specs=[
            pl.BlockSpec((1, S, D), lambda b, j: (b, j, 0)),
            pl.BlockSpec((1, 1, 1, S), lambda b, j: (b, j, 0, 0)),
            pl.BlockSpec((MAX_SEQ, D), lambda b, j: (0, 0)),
        ],
        out_specs=pl.BlockSpec((1, S, D), lambda b, j: (b, j, 0)),
        out_shape=jax.ShapeDtypeStruct((B, L, D), jnp.float32),
    )(x, pos4, pe[0])
    return out
